# baseline (speedup 1.0000x reference)
import functools

import jax
import jax.numpy as jnp
from jax import lax
from jax.experimental import pallas as pl
from jax.experimental.pallas import tpu as pltpu
from jax.experimental.pallas import tpu_sc as plsc

NUM_CORES = 2
NUM_SUBCORES = 16
NW = NUM_CORES * NUM_SUBCORES

B = 16384
D = 64
DV = 128
N_ROWS = 1_000_000
NV = N_ROWS // 2
HIDDEN = 128
CHUNK = 128
B_PER_W = B // NW
CH_PER_W = B_PER_W // CHUNK

TBM = 49152
TBQ = TBM // 4
NBLK = (N_ROWS + TBM - 1) // TBM
NV4 = NBLK * TBQ


_TDIMS = (((0,), (0,)), ((), ()))


def _pack_pair(a, b):
    ua = lax.bitcast_convert_type(a, jnp.uint16).astype(jnp.uint32)
    ub = lax.bitcast_convert_type(b, jnp.uint16).astype(jnp.uint32)
    return lax.bitcast_convert_type(
        jnp.bitwise_or(jnp.left_shift(ua, 16), ub), jnp.float32)


_SUB = 512


def _transpose_body(src_ref, eye_ref, dst_ref):
    e = eye_ref[...]
    for qa, col0 in ((0, 0), (2, D)):
        for c in range(0, TBQ, _SUB):
            a = lax.dot_general(
                src_ref[:, qa * TBQ + c:qa * TBQ + c + _SUB], e, _TDIMS,
                preferred_element_type=jnp.float32).astype(jnp.bfloat16)
            b = lax.dot_general(
                src_ref[:, (qa + 1) * TBQ + c:(qa + 1) * TBQ + c + _SUB], e,
                _TDIMS, preferred_element_type=jnp.float32).astype(jnp.bfloat16)
            dst_ref[pl.ds(c, _SUB), col0:col0 + D] = _pack_pair(a, b)


def _tc_transpose(tT, eye):
    return pl.pallas_call(
        _transpose_body,
        grid=(NBLK,),
        in_specs=[pl.BlockSpec((D, TBM), lambda i: (0, i)),
                  pl.BlockSpec((D, D), lambda i: (0, 0))],
        out_specs=pl.BlockSpec((TBQ, DV), lambda i: (i, 0)),
        out_shape=jax.ShapeDtypeStruct((NV4, DV), jnp.float32),
    )(tT, eye)


def _sc_gather(t2, vidx):
    mesh = plsc.VectorSubcoreMesh(core_axis_name="c", subcore_axis_name="s")

    @functools.partial(
        pl.kernel,
        out_type=jax.ShapeDtypeStruct((B, DV), jnp.float32),
        mesh=mesh,
        compiler_params=pltpu.CompilerParams(use_tc_tiling_on_sc=False),
        scratch_types=[
            pltpu.VMEM((B_PER_W,), jnp.int32),
            pltpu.VMEM((B_PER_W, DV), jnp.float32),
            pltpu.SemaphoreType.DMA,
        ],
    )
    def gather_kernel(t2_hbm, ix_hbm, out_hbm, ix_v, rows, sem):
        wid = lax.axis_index("s") * NUM_CORES + lax.axis_index("c")
        base = wid * B_PER_W
        pltpu.sync_copy(ix_hbm.at[pl.ds(base, B_PER_W)], ix_v)
        copies = []
        for j in range(CH_PER_W):
            sl = pl.ds(j * CHUNK, CHUNK)
            copies.append(
                pltpu.async_copy(t2_hbm.at[ix_v.at[sl]], rows.at[sl], sem))
        for c in copies:
            c.wait()
        pltpu.sync_copy(rows, out_hbm.at[pl.ds(base, B_PER_W)])

    return gather_kernel(t2, vidx)


def _unpack(rows, qt):
    half = jnp.where(qt >= 2, rows[:, D:], rows[:, :D])
    u = lax.bitcast_convert_type(half, jnp.uint32)
    hi = jnp.bitwise_and(u, jnp.uint32(0xFFFF0000))
    lo = jnp.left_shift(u, 16)
    bits = jnp.where(jnp.bitwise_and(qt, 1) == 1, lo, hi)
    return lax.bitcast_convert_type(bits, jnp.float32)


def _mlp_body(ue_ref, uq_ref, ie_ref, iq_ref,
              wu1, bu1, wu2, bu2, wi1, bi1, wi2, bi2,
              uo_ref, io_ref):
    _T2 = (((0,), (1,)), ((), ()))
    u = _unpack(ue_ref[...], uq_ref[...])
    hu = jnp.maximum(
        jnp.dot(u, wu1[...], preferred_element_type=jnp.float32) + bu1[...], 0.0)
    uo_ref[...] = lax.dot_general(
        wu2[...], hu, _T2, preferred_element_type=jnp.float32) + bu2[...]
    it = _unpack(ie_ref[...], iq_ref[...])
    hi = jnp.maximum(
        jnp.dot(it, wi1[...], preferred_element_type=jnp.float32) + bi1[...], 0.0)
    io_ref[...] = lax.dot_general(
        wi2[...], hi, _T2, preferred_element_type=jnp.float32) + bi2[...]


def _tc_mlp(ue, up, ie, ip, Wu1, bu1, Wu2, bu2, Wi1, bi1, Wi2, bi2):
    BM = 4096
    grid = (B // BM,)
    row_spec = pl.BlockSpec((BM, DV), lambda i: (i, 0))
    par_spec = pl.BlockSpec((BM, 1), lambda i: (i, 0))
    out_spec = pl.BlockSpec((D, BM), lambda i: (0, i))
    hid_w = pl.BlockSpec((D, HIDDEN), lambda i: (0, 0))
    out_w = pl.BlockSpec((HIDDEN, D), lambda i: (0, 0))
    hid_b = pl.BlockSpec((1, HIDDEN), lambda i: (0, 0))
    out_b = pl.BlockSpec((D, 1), lambda i: (0, 0))
    uoT, ioT = pl.pallas_call(
        _mlp_body,
        grid=grid,
        in_specs=[row_spec, par_spec, row_spec, par_spec,
                  hid_w, hid_b, out_w, out_b,
                  hid_w, hid_b, out_w, out_b],
        out_specs=[out_spec, out_spec],
        out_shape=[
            jax.ShapeDtypeStruct((D, B), jnp.float32),
            jax.ShapeDtypeStruct((D, B), jnp.float32),
        ],
    )(ue, up, ie, ip,
      Wu1, bu1.reshape(1, HIDDEN), Wu2, bu2.reshape(D, 1),
      Wi1, bi1.reshape(1, HIDDEN), Wi2, bi2.reshape(D, 1))
    return uoT.T, ioT.T


def kernel(user_input, item_input, user_table, item_table,
           Wu1, bu1, Wu2, bu2, Wi1, bi1, Wi2, bi2):
    def vmap_idx(r):
        return (r // TBM) * TBQ + r % TBQ

    def quarter(r):
        return jnp.bitwise_and(r // TBQ, 3)

    uvidx = vmap_idx(user_input)
    ividx = vmap_idx(item_input)
    up = quarter(user_input).reshape(B, 1)
    ip = quarter(item_input).reshape(B, 1)
    eye = jnp.eye(D, dtype=jnp.float32)
    ut2 = _tc_transpose(user_table.T, eye)
    ue = _sc_gather(ut2, uvidx)
    it2 = _tc_transpose(item_table.T, eye)
    ie = _sc_gather(it2, ividx)
    uo, io = _tc_mlp(ue, up, ie, ip, Wu1, bu1, Wu2, bu2, Wi1, bi1, Wi2, bi2)
    return (uo, io)


# speedup vs baseline: 1.0079x; 1.0079x over previous
import functools

import jax
import jax.numpy as jnp
from jax import lax
from jax.experimental import pallas as pl
from jax.experimental.pallas import tpu as pltpu
from jax.experimental.pallas import tpu_sc as plsc

NUM_CORES = 2
NUM_SUBCORES = 16
NW = NUM_CORES * NUM_SUBCORES

B = 16384
D = 64
DV = 128
N_ROWS = 1_000_000
NV = N_ROWS // 2
HIDDEN = 128
CHUNK = 128
B_PER_W = B // NW
CH_PER_W = B_PER_W // CHUNK

TBM = 32768
TBQ = TBM // 4
NBLK = (N_ROWS + TBM - 1) // TBM
NV4 = NBLK * TBQ


_TDIMS = (((0,), (0,)), ((), ()))


def _pack_pair(a, b):
    ua = lax.bitcast_convert_type(a, jnp.uint16).astype(jnp.uint32)
    ub = lax.bitcast_convert_type(b, jnp.uint16).astype(jnp.uint32)
    return lax.bitcast_convert_type(
        jnp.bitwise_or(jnp.left_shift(ua, 16), ub), jnp.float32)


_SUB = 512


def _transpose_body(src_ref, eye_ref, dst_ref):
    e = eye_ref[...]
    for qa, col0 in ((0, 0), (2, D)):
        for c in range(0, TBQ, _SUB):
            a = lax.dot_general(
                src_ref[:, qa * TBQ + c:qa * TBQ + c + _SUB], e, _TDIMS,
                preferred_element_type=jnp.float32).astype(jnp.bfloat16)
            b = lax.dot_general(
                src_ref[:, (qa + 1) * TBQ + c:(qa + 1) * TBQ + c + _SUB], e,
                _TDIMS, preferred_element_type=jnp.float32).astype(jnp.bfloat16)
            dst_ref[pl.ds(c, _SUB), col0:col0 + D] = _pack_pair(a, b)


def _tc_transpose(tT, eye):
    return pl.pallas_call(
        _transpose_body,
        grid=(NBLK,),
        in_specs=[pl.BlockSpec((D, TBM), lambda i: (0, i)),
                  pl.BlockSpec((D, D), lambda i: (0, 0))],
        out_specs=pl.BlockSpec((TBQ, DV), lambda i: (i, 0)),
        out_shape=jax.ShapeDtypeStruct((NV4, DV), jnp.float32),
    )(tT, eye)


def _sc_gather(t2, vidx):
    mesh = plsc.VectorSubcoreMesh(core_axis_name="c", subcore_axis_name="s")

    @functools.partial(
        pl.kernel,
        out_type=jax.ShapeDtypeStruct((B, DV), jnp.float32),
        mesh=mesh,
        compiler_params=pltpu.CompilerParams(use_tc_tiling_on_sc=False),
        scratch_types=[
            pltpu.VMEM((B_PER_W,), jnp.int32),
            pltpu.VMEM((B_PER_W, DV), jnp.float32),
            pltpu.SemaphoreType.DMA,
        ],
    )
    def gather_kernel(t2_hbm, ix_hbm, out_hbm, ix_v, rows, sem):
        wid = lax.axis_index("s") * NUM_CORES + lax.axis_index("c")
        base = wid * B_PER_W
        pltpu.sync_copy(ix_hbm.at[pl.ds(base, B_PER_W)], ix_v)
        copies = []
        for j in range(CH_PER_W):
            sl = pl.ds(j * CHUNK, CHUNK)
            copies.append(
                pltpu.async_copy(t2_hbm.at[ix_v.at[sl]], rows.at[sl], sem))
        for c in copies:
            c.wait()
        pltpu.sync_copy(rows, out_hbm.at[pl.ds(base, B_PER_W)])

    return gather_kernel(t2, vidx)


def _unpack(rows, qt):
    half = jnp.where(qt >= 2, rows[:, D:], rows[:, :D])
    u = lax.bitcast_convert_type(half, jnp.uint32)
    hi = jnp.bitwise_and(u, jnp.uint32(0xFFFF0000))
    lo = jnp.left_shift(u, 16)
    bits = jnp.where(jnp.bitwise_and(qt, 1) == 1, lo, hi)
    return lax.bitcast_convert_type(bits, jnp.float32)


def _mlp_body(e_ref, q_ref, w1, b1, w2, b2, o_ref):
    _T2 = (((0,), (1,)), ((), ()))
    x = _unpack(e_ref[...], q_ref[...])
    h = jnp.maximum(
        jnp.dot(x, w1[...], preferred_element_type=jnp.float32) + b1[...], 0.0)
    o_ref[...] = lax.dot_general(
        w2[...], h, _T2, preferred_element_type=jnp.float32) + b2[...]


def _tc_mlp_tower(e, q, W1, b1, W2, b2):
    BM = 4096
    grid = (B // BM,)
    oT = pl.pallas_call(
        _mlp_body,
        grid=grid,
        in_specs=[pl.BlockSpec((BM, DV), lambda i: (i, 0)),
                  pl.BlockSpec((BM, 1), lambda i: (i, 0)),
                  pl.BlockSpec((D, HIDDEN), lambda i: (0, 0)),
                  pl.BlockSpec((1, HIDDEN), lambda i: (0, 0)),
                  pl.BlockSpec((HIDDEN, D), lambda i: (0, 0)),
                  pl.BlockSpec((D, 1), lambda i: (0, 0))],
        out_specs=pl.BlockSpec((D, BM), lambda i: (0, i)),
        out_shape=jax.ShapeDtypeStruct((D, B), jnp.float32),
    )(e, q, W1, b1.reshape(1, HIDDEN), W2, b2.reshape(D, 1))
    return oT.T


def kernel(user_input, item_input, user_table, item_table,
           Wu1, bu1, Wu2, bu2, Wi1, bi1, Wi2, bi2):
    def vmap_idx(r):
        return (r // TBM) * TBQ + r % TBQ

    def quarter(r):
        return jnp.bitwise_and(r // TBQ, 3)

    uvidx = vmap_idx(user_input)
    ividx = vmap_idx(item_input)
    up = quarter(user_input).reshape(B, 1)
    ip = quarter(item_input).reshape(B, 1)
    eye = jnp.eye(D, dtype=jnp.float32)
    ut2 = _tc_transpose(user_table.T, eye)
    ue = _sc_gather(ut2, uvidx)
    it2 = _tc_transpose(item_table.T, eye)
    ie = _sc_gather(it2, ividx)
    uo = _tc_mlp_tower(ue, up, Wu1, bu1, Wu2, bu2)
    io = _tc_mlp_tower(ie, ip, Wi1, bi1, Wi2, bi2)
    return (uo, io)


# speedup vs baseline: 1.0151x; 1.0071x over previous
import functools

import jax
import jax.numpy as jnp
from jax import lax
from jax.experimental import pallas as pl
from jax.experimental.pallas import tpu as pltpu
from jax.experimental.pallas import tpu_sc as plsc

NUM_CORES = 2
NUM_SUBCORES = 16
NW = NUM_CORES * NUM_SUBCORES

B = 16384
D = 64
DV = 128
N_ROWS = 1_000_000
NV = N_ROWS // 2
HIDDEN = 128
CHUNK = 128
B_PER_W = B // NW
CH_PER_W = B_PER_W // CHUNK

TBM = 32768
TBQ = TBM // 4
NBLK = (N_ROWS + TBM - 1) // TBM
NV4 = NBLK * TBQ


_TDIMS = (((0,), (0,)), ((), ()))


def _pack_pair(a, b):
    ua = lax.bitcast_convert_type(a, jnp.uint16).astype(jnp.uint32)
    ub = lax.bitcast_convert_type(b, jnp.uint16).astype(jnp.uint32)
    return lax.bitcast_convert_type(
        jnp.bitwise_or(jnp.left_shift(ua, 16), ub), jnp.float32)


_SUB = 512


def _transpose_body(src_ref, eye_ref, dst_ref):
    e = eye_ref[...]
    for qa, col0 in ((0, 0), (2, D)):
        for c in range(0, TBQ, _SUB):
            a = lax.dot_general(
                src_ref[:, qa * TBQ + c:qa * TBQ + c + _SUB], e, _TDIMS,
                preferred_element_type=jnp.float32).astype(jnp.bfloat16)
            b = lax.dot_general(
                src_ref[:, (qa + 1) * TBQ + c:(qa + 1) * TBQ + c + _SUB], e,
                _TDIMS, preferred_element_type=jnp.float32).astype(jnp.bfloat16)
            dst_ref[pl.ds(c, _SUB), col0:col0 + D] = _pack_pair(a, b)


def _tc_transpose(tT, eye):
    return pl.pallas_call(
        _transpose_body,
        grid=(NBLK,),
        in_specs=[pl.BlockSpec((D, TBM), lambda i: (0, i)),
                  pl.BlockSpec((D, D), lambda i: (0, 0))],
        out_specs=pl.BlockSpec((TBQ, DV), lambda i: (i, 0)),
        out_shape=jax.ShapeDtypeStruct((NV4, DV), jnp.float32),
    )(tT, eye)


def _sc_gather(t2, vidx):
    mesh = plsc.VectorSubcoreMesh(core_axis_name="c", subcore_axis_name="s")

    @functools.partial(
        pl.kernel,
        out_type=jax.ShapeDtypeStruct((B, DV), jnp.float32),
        mesh=mesh,
        compiler_params=pltpu.CompilerParams(use_tc_tiling_on_sc=False),
        scratch_types=[
            pltpu.VMEM((B_PER_W,), jnp.int32),
            pltpu.VMEM((B_PER_W, DV), jnp.float32),
            pltpu.SemaphoreType.DMA,
        ],
    )
    def gather_kernel(t2_hbm, ix_hbm, out_hbm, ix_v, rows, sem):
        wid = lax.axis_index("s") * NUM_CORES + lax.axis_index("c")
        base = wid * B_PER_W
        pltpu.sync_copy(ix_hbm.at[pl.ds(base, B_PER_W)], ix_v)
        copies = []
        for j in range(CH_PER_W):
            sl = pl.ds(j * CHUNK, CHUNK)
            copies.append(
                pltpu.async_copy(t2_hbm.at[ix_v.at[sl]], rows.at[sl], sem))
        for c in copies:
            c.wait()
        pltpu.sync_copy(rows, out_hbm.at[pl.ds(base, B_PER_W)])

    return gather_kernel(t2, vidx)


def _unpack(rows, qt):
    half = jnp.where(qt >= 2, rows[:, D:], rows[:, :D])
    u = lax.bitcast_convert_type(half, jnp.uint32)
    hi = jnp.bitwise_and(u, jnp.uint32(0xFFFF0000))
    lo = jnp.left_shift(u, 16)
    bits = jnp.where(jnp.bitwise_and(qt, 1) == 1, lo, hi)
    return lax.bitcast_convert_type(bits, jnp.float32)


def _mlp_body(ue_ref, uq_ref, ie_ref, iq_ref,
              wu1, bu1, wu2, bu2, wi1, bi1, wi2, bi2,
              uo_ref, io_ref):
    _T2 = (((0,), (1,)), ((), ()))
    u = _unpack(ue_ref[...], uq_ref[...])
    hu = jnp.maximum(
        jnp.dot(u, wu1[...], preferred_element_type=jnp.float32) + bu1[...], 0.0)
    uo_ref[...] = lax.dot_general(
        wu2[...], hu, _T2, preferred_element_type=jnp.float32) + bu2[...]
    it = _unpack(ie_ref[...], iq_ref[...])
    hi = jnp.maximum(
        jnp.dot(it, wi1[...], preferred_element_type=jnp.float32) + bi1[...], 0.0)
    io_ref[...] = lax.dot_general(
        wi2[...], hi, _T2, preferred_element_type=jnp.float32) + bi2[...]


def _tc_mlp(ue, up, ie, ip, Wu1, bu1, Wu2, bu2, Wi1, bi1, Wi2, bi2):
    BM = 4096
    grid = (B // BM,)
    row_spec = pl.BlockSpec((BM, DV), lambda i: (i, 0))
    par_spec = pl.BlockSpec((BM, 1), lambda i: (i, 0))
    out_spec = pl.BlockSpec((D, BM), lambda i: (0, i))
    hid_w = pl.BlockSpec((D, HIDDEN), lambda i: (0, 0))
    out_w = pl.BlockSpec((HIDDEN, D), lambda i: (0, 0))
    hid_b = pl.BlockSpec((1, HIDDEN), lambda i: (0, 0))
    out_b = pl.BlockSpec((D, 1), lambda i: (0, 0))
    uoT, ioT = pl.pallas_call(
        _mlp_body,
        grid=grid,
        in_specs=[row_spec, par_spec, row_spec, par_spec,
                  hid_w, hid_b, out_w, out_b,
                  hid_w, hid_b, out_w, out_b],
        out_specs=[out_spec, out_spec],
        out_shape=[
            jax.ShapeDtypeStruct((D, B), jnp.float32),
            jax.ShapeDtypeStruct((D, B), jnp.float32),
        ],
    )(ue, up, ie, ip,
      Wu1, bu1.reshape(1, HIDDEN), Wu2, bu2.reshape(D, 1),
      Wi1, bi1.reshape(1, HIDDEN), Wi2, bi2.reshape(D, 1))
    return uoT.T, ioT.T


def kernel(user_input, item_input, user_table, item_table,
           Wu1, bu1, Wu2, bu2, Wi1, bi1, Wi2, bi2):
    def vmap_idx(r):
        return (r // TBM) * TBQ + r % TBQ

    def quarter(r):
        return jnp.bitwise_and(r // TBQ, 3)

    uvidx = vmap_idx(user_input)
    ividx = vmap_idx(item_input)
    up = quarter(user_input).reshape(B, 1)
    ip = quarter(item_input).reshape(B, 1)
    eye = jnp.eye(D, dtype=jnp.float32)
    ut2 = _tc_transpose(user_table.T, eye)
    ue = _sc_gather(ut2, uvidx)
    it2 = _tc_transpose(item_table.T, eye)
    ie = _sc_gather(it2, ividx)
    uo, io = _tc_mlp(ue, up, ie, ip, Wu1, bu1, Wu2, bu2, Wi1, bi1, Wi2, bi2)
    return (uo, io)


# speedup vs baseline: 1.0260x; 1.0108x over previous
import functools

import jax
import jax.numpy as jnp
from jax import lax
from jax.experimental import pallas as pl
from jax.experimental.pallas import tpu as pltpu
from jax.experimental.pallas import tpu_sc as plsc

NUM_CORES = 2
NUM_SUBCORES = 16
NW = NUM_CORES * NUM_SUBCORES

B = 16384
D = 64
DV = 128
N_ROWS = 1_000_000
NV = N_ROWS // 2
HIDDEN = 128
CHUNK = 128
B_PER_W = B // NW
CH_PER_W = B_PER_W // CHUNK

TBM = 32768
TBQ = TBM // 4
NBLK = (N_ROWS + TBM - 1) // TBM
NV4 = NBLK * TBQ


_TDIMS = (((0,), (0,)), ((), ()))


def _pack_pair(a, b):
    ua = lax.bitcast_convert_type(a, jnp.uint16).astype(jnp.uint32)
    ub = lax.bitcast_convert_type(b, jnp.uint16).astype(jnp.uint32)
    return lax.bitcast_convert_type(
        jnp.bitwise_or(jnp.left_shift(ua, 16), ub), jnp.float32)


_SUB = 512


def _transpose_one(src_ref, e, dst_ref):
    for qa, col0 in ((0, 0), (2, D)):
        for c in range(0, TBQ, _SUB):
            a = lax.dot_general(
                src_ref[:, qa * TBQ + c:qa * TBQ + c + _SUB], e, _TDIMS,
                preferred_element_type=jnp.float32).astype(jnp.bfloat16)
            b = lax.dot_general(
                src_ref[:, (qa + 1) * TBQ + c:(qa + 1) * TBQ + c + _SUB], e,
                _TDIMS, preferred_element_type=jnp.float32).astype(jnp.bfloat16)
            dst_ref[pl.ds(c, _SUB), col0:col0 + D] = _pack_pair(a, b)


def _transpose_body(srcu_ref, srci_ref, eye_ref, dstu_ref, dsti_ref):
    e = eye_ref[...]
    _transpose_one(srcu_ref, e, dstu_ref)
    _transpose_one(srci_ref, e, dsti_ref)


def _tc_transpose2(utT, itT, eye):
    src_spec = pl.BlockSpec((D, TBM), lambda i: (0, i))
    dst_spec = pl.BlockSpec((TBQ, DV), lambda i: (i, 0))
    return pl.pallas_call(
        _transpose_body,
        grid=(NBLK,),
        in_specs=[src_spec, src_spec, pl.BlockSpec((D, D), lambda i: (0, 0))],
        out_specs=[dst_spec, dst_spec],
        out_shape=[jax.ShapeDtypeStruct((NV4, DV), jnp.float32),
                   jax.ShapeDtypeStruct((NV4, DV), jnp.float32)],
    )(utT, itT, eye)


def _sc_gather(t2, vidx):
    mesh = plsc.VectorSubcoreMesh(core_axis_name="c", subcore_axis_name="s")

    @functools.partial(
        pl.kernel,
        out_type=jax.ShapeDtypeStruct((B, DV), jnp.float32),
        mesh=mesh,
        compiler_params=pltpu.CompilerParams(use_tc_tiling_on_sc=False),
        scratch_types=[
            pltpu.VMEM((B_PER_W,), jnp.int32),
            pltpu.VMEM((B_PER_W, DV), jnp.float32),
            pltpu.SemaphoreType.DMA,
        ],
    )
    def gather_kernel(t2_hbm, ix_hbm, out_hbm, ix_v, rows, sem):
        wid = lax.axis_index("s") * NUM_CORES + lax.axis_index("c")
        base = wid * B_PER_W
        pltpu.sync_copy(ix_hbm.at[pl.ds(base, B_PER_W)], ix_v)
        copies = []
        for j in range(CH_PER_W):
            sl = pl.ds(j * CHUNK, CHUNK)
            copies.append(
                pltpu.async_copy(t2_hbm.at[ix_v.at[sl]], rows.at[sl], sem))
        for c in copies:
            c.wait()
        pltpu.sync_copy(rows, out_hbm.at[pl.ds(base, B_PER_W)])

    return gather_kernel(t2, vidx)


def _unpack(rows, qt):
    half = jnp.where(qt >= 2, rows[:, D:], rows[:, :D])
    u = lax.bitcast_convert_type(half, jnp.uint32)
    hi = jnp.bitwise_and(u, jnp.uint32(0xFFFF0000))
    lo = jnp.left_shift(u, 16)
    bits = jnp.where(jnp.bitwise_and(qt, 1) == 1, lo, hi)
    return lax.bitcast_convert_type(bits, jnp.float32)


def _mlp_body(ue_ref, uq_ref, ie_ref, iq_ref,
              wu1, bu1, wu2, bu2, wi1, bi1, wi2, bi2,
              uo_ref, io_ref):
    _T2 = (((0,), (1,)), ((), ()))
    u = _unpack(ue_ref[...], uq_ref[...])
    hu = jnp.maximum(
        jnp.dot(u, wu1[...], preferred_element_type=jnp.float32) + bu1[...], 0.0)
    uo_ref[...] = lax.dot_general(
        wu2[...], hu, _T2, preferred_element_type=jnp.float32) + bu2[...]
    it = _unpack(ie_ref[...], iq_ref[...])
    hi = jnp.maximum(
        jnp.dot(it, wi1[...], preferred_element_type=jnp.float32) + bi1[...], 0.0)
    io_ref[...] = lax.dot_general(
        wi2[...], hi, _T2, preferred_element_type=jnp.float32) + bi2[...]


def _tc_mlp(ue, up, ie, ip, Wu1, bu1, Wu2, bu2, Wi1, bi1, Wi2, bi2):
    BM = 4096
    grid = (B // BM,)
    row_spec = pl.BlockSpec((BM, DV), lambda i: (i, 0))
    par_spec = pl.BlockSpec((BM, 1), lambda i: (i, 0))
    out_spec = pl.BlockSpec((D, BM), lambda i: (0, i))
    hid_w = pl.BlockSpec((D, HIDDEN), lambda i: (0, 0))
    out_w = pl.BlockSpec((HIDDEN, D), lambda i: (0, 0))
    hid_b = pl.BlockSpec((1, HIDDEN), lambda i: (0, 0))
    out_b = pl.BlockSpec((D, 1), lambda i: (0, 0))
    uoT, ioT = pl.pallas_call(
        _mlp_body,
        grid=grid,
        in_specs=[row_spec, par_spec, row_spec, par_spec,
                  hid_w, hid_b, out_w, out_b,
                  hid_w, hid_b, out_w, out_b],
        out_specs=[out_spec, out_spec],
        out_shape=[
            jax.ShapeDtypeStruct((D, B), jnp.float32),
            jax.ShapeDtypeStruct((D, B), jnp.float32),
        ],
    )(ue, up, ie, ip,
      Wu1, bu1.reshape(1, HIDDEN), Wu2, bu2.reshape(D, 1),
      Wi1, bi1.reshape(1, HIDDEN), Wi2, bi2.reshape(D, 1))
    return uoT.T, ioT.T


def kernel(user_input, item_input, user_table, item_table,
           Wu1, bu1, Wu2, bu2, Wi1, bi1, Wi2, bi2):
    def vmap_idx(r):
        return (r // TBM) * TBQ + r % TBQ

    def quarter(r):
        return jnp.bitwise_and(r // TBQ, 3)

    uvidx = vmap_idx(user_input)
    ividx = vmap_idx(item_input)
    up = quarter(user_input).reshape(B, 1)
    ip = quarter(item_input).reshape(B, 1)
    eye = jnp.eye(D, dtype=jnp.float32)
    ut2, it2 = _tc_transpose2(user_table.T, item_table.T, eye)
    ue = _sc_gather(ut2, uvidx)
    ie = _sc_gather(it2, ividx)
    uo, io = _tc_mlp(ue, up, ie, ip, Wu1, bu1, Wu2, bu2, Wi1, bi1, Wi2, bi2)
    return (uo, io)
